# baseline (speedup 1.0000x reference)
import functools

import jax
import jax.numpy as jnp
from jax import lax
from jax.experimental import pallas as pl
from jax.experimental.pallas import tpu as pltpu
from jax.experimental.pallas import tpu_sc as plsc

NC = 2
NS = 16
NW = NC * NS
CH = 128
ZR = 32


def _fill_f32(ref, rows, cols, val):
    v = jnp.full((16,), val, jnp.float32)
    for r in range(rows):
        for j in range(cols // 16):
            ref[r, pl.ds(j * 16, 16)] = v


def _make_agg(n_acc, d, e_pad, ch=CH):
    epw = e_pad // NW
    n_chunks = epw // ch
    rpt = n_acc // NS
    nz = rpt // ZR
    mesh = plsc.VectorSubcoreMesh(core_axis_name="c", subcore_axis_name="s")

    assert n_chunks % 2 == 0
    out_type = jax.ShapeDtypeStruct((NC * n_acc, d), jnp.float32)
    scratch = [
        pltpu.VMEM((ch,), jnp.int32),
        pltpu.VMEM((ch,), jnp.int32),
        pltpu.VMEM((ch, d), jnp.float32),
        pltpu.VMEM((ch,), jnp.int32),
        pltpu.VMEM((ch,), jnp.int32),
        pltpu.VMEM((ch, d), jnp.float32),
        pltpu.VMEM((ZR, d), jnp.float32),
        pltpu.VMEM_SHARED((n_acc, d), jnp.float32),
        pltpu.SemaphoreType.DMA,
        pltpu.SemaphoreType.DMA,
    ]

    def body(table, src, dst, out_agg, src_v0, dst_v0, rows_v0, src_v1,
             dst_v1, rows_v1, zrow_v, agg_sh, sem0, sem1):
        c = lax.axis_index("c")
        s = lax.axis_index("s")
        wid = s * NC + c
        bufs = ((src_v0, dst_v0, rows_v0, sem0),
                (src_v1, dst_v1, rows_v1, sem1))

        def prefetch(j, b):
            src_v, dst_v, rows_v, sem = bufs[b]
            base = wid * epw + j * ch
            pltpu.sync_copy(src.at[pl.ds(base, ch)], src_v)
            pltpu.sync_copy(dst.at[pl.ds(base, ch)], dst_v)
            pltpu.async_copy(table.at[src_v], rows_v, sem)

        def finish(b):
            src_v, dst_v, rows_v, sem = bufs[b]
            pltpu.make_async_copy(table.at[src_v], rows_v, sem).wait()
            pltpu.sync_copy(rows_v, agg_sh.at[dst_v], add=True)

        _fill_f32(zrow_v, ZR, d, 0.0)

        for z in range(nz):
            r0 = (s * nz + z) * ZR
            pltpu.sync_copy(zrow_v, agg_sh.at[pl.ds(r0, ZR)])
        plsc.subcore_barrier()

        prefetch(0, 0)
        prefetch(1, 1)

        def pair(p, _):
            j = 2 * p
            finish(0)

            @pl.when(j + 2 < n_chunks)
            def _():
                prefetch(j + 2, 0)

            finish(1)

            @pl.when(j + 3 < n_chunks)
            def _():
                prefetch(j + 3, 1)

            return 0

        lax.fori_loop(0, n_chunks // 2, pair, 0, unroll=False)
        plsc.subcore_barrier()

        r0 = s * rpt
        pltpu.sync_copy(agg_sh.at[pl.ds(r0, rpt)],
                        out_agg.at[pl.ds(c * n_acc + r0, rpt)])

    return pl.kernel(body, out_type=out_type, mesh=mesh,
                     scratch_types=scratch)


def _make_cnt(n_acc, e_pad, cw=128, ch=CH):
    epw = e_pad // NW
    n_chunks = epw // ch
    rpt = n_acc // NS
    nz = rpt // ZR
    mesh = plsc.VectorSubcoreMesh(core_axis_name="c", subcore_axis_name="s")

    def body(dst, out_cnt, dst_v, ones_v, zcnt_v, cnt_sh):
        c = lax.axis_index("c")
        s = lax.axis_index("s")
        wid = s * NC + c

        _fill_f32(ones_v, ch, cw, 1.0)
        _fill_f32(zcnt_v, ZR, cw, 0.0)
        for z in range(nz):
            pltpu.sync_copy(zcnt_v, cnt_sh.at[pl.ds((s * nz + z) * ZR, ZR)])
        plsc.subcore_barrier()

        def chunk(j, _):
            base = wid * epw + j * ch
            pltpu.sync_copy(dst.at[pl.ds(base, ch)], dst_v)
            pltpu.sync_copy(ones_v, cnt_sh.at[dst_v], add=True)
            return 0

        lax.fori_loop(0, n_chunks, chunk, 0, unroll=False)
        plsc.subcore_barrier()

        r0 = s * rpt
        pltpu.sync_copy(cnt_sh.at[pl.ds(r0, rpt)],
                        out_cnt.at[pl.ds(c * n_acc + r0, rpt)])

    return pl.kernel(
        body,
        out_type=jax.ShapeDtypeStruct((NC * n_acc, cw), jnp.float32),
        mesh=mesh,
        scratch_types=[
            pltpu.VMEM((ch,), jnp.int32),
            pltpu.VMEM((ch, cw), jnp.float32),
            pltpu.VMEM((ZR, cw), jnp.float32),
            pltpu.VMEM_SHARED((n_acc, cw), jnp.float32),
        ],
    )


def _make_pair_gather(d, b_pad):
    bpw = b_pad // NW
    n_chunks = bpw // CH
    mesh = plsc.VectorSubcoreMesh(core_axis_name="c", subcore_axis_name="s")

    assert n_chunks % 2 == 0

    def body(table, idx, out, idx_v0, rows_v0, idx_v1, rows_v1, sem0, sem1):
        c = lax.axis_index("c")
        s = lax.axis_index("s")
        wid = s * NC + c
        bufs = ((idx_v0, rows_v0, sem0), (idx_v1, rows_v1, sem1))

        def prefetch(j, b):
            idx_v, rows_v, sem = bufs[b]
            pltpu.sync_copy(idx.at[pl.ds(wid * bpw + j * CH, CH)], idx_v)
            pltpu.async_copy(table.at[idx_v], rows_v, sem)

        def finish(j, b):
            idx_v, rows_v, sem = bufs[b]
            pltpu.make_async_copy(table.at[idx_v], rows_v, sem).wait()
            pltpu.sync_copy(rows_v, out.at[pl.ds(wid * bpw + j * CH, CH)])

        prefetch(0, 0)
        prefetch(1, 1)

        def pair(p, _):
            j = 2 * p
            finish(j, 0)

            @pl.when(j + 2 < n_chunks)
            def _():
                prefetch(j + 2, 0)

            finish(j + 1, 1)

            @pl.when(j + 3 < n_chunks)
            def _():
                prefetch(j + 3, 1)

            return 0

        lax.fori_loop(0, n_chunks // 2, pair, 0, unroll=False)

    return pl.kernel(
        body,
        out_type=jax.ShapeDtypeStruct((b_pad, d), jnp.float32),
        mesh=mesh,
        scratch_types=[
            pltpu.VMEM((CH,), jnp.int32),
            pltpu.VMEM((CH, d), jnp.float32),
            pltpu.VMEM((CH,), jnp.int32),
            pltpu.VMEM((CH, d), jnp.float32),
            pltpu.SemaphoreType.DMA,
            pltpu.SemaphoreType.DMA,
        ],
    )


def _node1_body(aggp, cntp, x, wlt, wrt, bl, out):
    cnt = cntp[0, :, 0:1] + cntp[1, :, 0:1]
    mean = (aggp[0] + aggp[1]) / jnp.maximum(cnt, 1.0)
    h = (jnp.dot(mean, wlt[...], preferred_element_type=jnp.float32)
         + jnp.dot(x[...], wrt[...], preferred_element_type=jnp.float32)
         + bl[...])
    out[...] = jnp.maximum(h, 0.0)


def _node2_body(aggp, cntp, h1, wlt, wrt, bl, p1t, p1b, p2t, p2b,
                m1at, m1bt, m1b, out):
    cnt = cntp[0, :, 0:1] + cntp[1, :, 0:1]
    mean = (aggp[0] + aggp[1]) / jnp.maximum(cnt, 1.0)
    h2 = (jnp.dot(mean, wlt[...], preferred_element_type=jnp.float32)
          + jnp.dot(h1[...], wrt[...], preferred_element_type=jnp.float32)
          + bl[...])
    h2 = jnp.maximum(h2, 0.0)
    hp = jnp.dot(h2, p1t[...], preferred_element_type=jnp.float32) + p1b[...]
    hp = jnp.dot(hp, p2t[...], preferred_element_type=jnp.float32) + p2b[...]
    out[0] = jnp.dot(hp, m1at[...], preferred_element_type=jnp.float32) + m1b[...]
    out[1] = jnp.dot(hp, m1bt[...], preferred_element_type=jnp.float32)


def _edge_body(g, m2t, m2b, m3t, m3b, m4r, m4b, out):
    e = jnp.maximum(g[0] + g[1], 0.0)
    e = jnp.maximum(
        jnp.dot(e, m2t[...], preferred_element_type=jnp.float32) + m2b[...], 0.0)
    e = jnp.maximum(
        jnp.dot(e, m3t[...], preferred_element_type=jnp.float32) + m3b[...], 0.0)
    o = jnp.sum(e * m4r[...], axis=1, keepdims=True) + m4b[...]
    out[...] = jnp.maximum(o, 0.0)


def _round_up(a, b):
    return (a + b - 1) // b * b


def kernel(x, edge_attr, edge_index, predict_edge_index, Wl1, bl1, Wr1,
           Wl2, bl2, Wr2, P1, p1b, P2, p2b, M1, m1b, M2, m2b, M3, m3b,
           M4, m4b):
    del edge_attr
    n, d = x.shape
    h = Wl1.shape[0]
    o = P2.shape[0]
    e = edge_index.shape[1]
    ep = predict_edge_index.shape[1]
    f32 = jnp.float32

    n_acc = _round_up(n + 1, NS * ZR)
    e_pad = _round_up(e, NW * 96 * 2)
    ep_pad = _round_up(ep, NW * CH)
    b_pad = 2 * ep_pad

    x_pad = jnp.pad(x, ((0, n_acc - n), (0, 0)))
    src = jnp.pad(edge_index[0], (0, e_pad - e))
    dst = jnp.pad(edge_index[1], (0, e_pad - e), constant_values=n)
    pe0 = jnp.pad(predict_edge_index[0], (0, ep_pad - ep))
    pe1 = jnp.pad(predict_edge_index[1], (0, ep_pad - ep))
    pair_idx = jnp.concatenate([pe0, pe1 + n_acc])

    wl1t = Wl1.T
    wr1t = Wr1.T
    wl2t = Wl2.T
    wr2t = Wr2.T
    p1t = P1.T
    p2t = P2.T
    m1at = M1[:, :o].T
    m1bt = M1[:, o:].T
    m2t = M2.T
    m3t = M3.T
    m4r = M4
    bl1r = bl1.reshape(1, h)
    bl2r = bl2.reshape(1, h)
    p1br = p1b.reshape(1, h)
    p2br = p2b.reshape(1, o)
    m1br = m1b.reshape(1, o)
    m2br = m2b.reshape(1, o)
    m3br = m3b.reshape(1, o)
    m4br = m4b.reshape(1, 1)

    aggp1 = _make_agg(n_acc, d, e_pad, ch=96)(x, src, dst)
    aggp1 = aggp1.reshape(NC, n_acc, d)
    cntp = _make_cnt(n_acc, e_pad, ch=96)(dst)
    cntp = cntp.reshape(NC, n_acc, 128)

    br = 2048
    grid_n = n_acc // br
    wspec = pl.BlockSpec((d, h), lambda i: (0, 0))
    bspec = lambda cols: pl.BlockSpec((1, cols), lambda i: (0, 0))
    h1 = pl.pallas_call(
        _node1_body,
        grid=(grid_n,),
        in_specs=[
            pl.BlockSpec((2, br, d), lambda i: (0, i, 0)),
            pl.BlockSpec((2, br, 128), lambda i: (0, i, 0)),
            pl.BlockSpec((br, d), lambda i: (i, 0)),
            wspec, wspec, bspec(h),
        ],
        out_specs=pl.BlockSpec((br, h), lambda i: (i, 0)),
        out_shape=jax.ShapeDtypeStruct((n_acc, h), f32),
    )(aggp1, cntp, x_pad, wl1t, wr1t, bl1r)

    aggp2 = _make_agg(n_acc, h, e_pad, ch=96)(h1, src, dst)
    aggp2 = aggp2.reshape(NC, n_acc, h)

    ab = pl.pallas_call(
        _node2_body,
        grid=(grid_n,),
        in_specs=[
            pl.BlockSpec((2, br, h), lambda i: (0, i, 0)),
            pl.BlockSpec((2, br, 128), lambda i: (0, i, 0)),
            pl.BlockSpec((br, h), lambda i: (i, 0)),
            wspec, wspec, bspec(h),
            wspec, bspec(h), wspec, bspec(o),
            wspec, wspec, bspec(o),
        ],
        out_specs=pl.BlockSpec((2, br, o), lambda i: (0, i, 0)),
        out_shape=jax.ShapeDtypeStruct((2, n_acc, o), f32),
    )(aggp2, cntp, h1, wl2t, wr2t, bl2r, p1t, p1br, p2t, p2br,
      m1at, m1bt, m1br)
    tab = ab.reshape(2 * n_acc, o)

    g = _make_pair_gather(o, b_pad)(tab, pair_idx)
    g2 = g.reshape(2, ep_pad, o)

    bre = 2048
    grid_e = ep_pad // bre
    espec = pl.BlockSpec((o, o), lambda i: (0, 0))
    ebspec = lambda cols: pl.BlockSpec((1, cols), lambda i: (0, 0))
    out = pl.pallas_call(
        _edge_body,
        grid=(grid_e,),
        in_specs=[
            pl.BlockSpec((2, bre, o), lambda i: (0, i, 0)),
            espec, ebspec(o), espec, ebspec(o), ebspec(o), ebspec(1),
        ],
        out_specs=pl.BlockSpec((bre, 1), lambda i: (i, 0)),
        out_shape=jax.ShapeDtypeStruct((ep_pad, 1), f32),
    )(g2, m2t, m2br, m3t, m3br, m4r, m4br)

    return out[:ep]


# speedup vs baseline: 2.0052x; 2.0052x over previous
import functools

import jax
import jax.numpy as jnp
from jax import lax
from jax.experimental import pallas as pl
from jax.experimental.pallas import tpu as pltpu
from jax.experimental.pallas import tpu_sc as plsc

NC = 2
NS = 16
NW = NC * NS
CH = 128
ZR = 32


def _fill_f32(ref, rows, cols, val):
    v = jnp.full((16,), val, jnp.float32)
    for r in range(rows):
        for j in range(cols // 16):
            ref[r, pl.ds(j * 16, 16)] = v


def _make_agg(n_acc, d, e_pad, ch=CH):
    epw = e_pad // NW
    n_chunks = epw // ch
    rpt = n_acc // NS
    nz = rpt // ZR
    mesh = plsc.VectorSubcoreMesh(core_axis_name="c", subcore_axis_name="s")

    assert n_chunks % 2 == 0
    out_type = jax.ShapeDtypeStruct((NC * n_acc, d), jnp.float32)
    scratch = [
        pltpu.VMEM((ch,), jnp.int32),
        pltpu.VMEM((ch,), jnp.int32),
        pltpu.VMEM((ch, d), jnp.float32),
        pltpu.VMEM((ch,), jnp.int32),
        pltpu.VMEM((ch,), jnp.int32),
        pltpu.VMEM((ch, d), jnp.float32),
        pltpu.VMEM((ZR, d), jnp.float32),
        pltpu.VMEM_SHARED((n_acc, d), jnp.float32),
        pltpu.SemaphoreType.DMA,
        pltpu.SemaphoreType.DMA,
    ]

    def body(table, src, dst, out_agg, src_v0, dst_v0, rows_v0, src_v1,
             dst_v1, rows_v1, zrow_v, agg_sh, sem0, sem1):
        c = lax.axis_index("c")
        s = lax.axis_index("s")
        wid = s * NC + c
        bufs = ((src_v0, dst_v0, rows_v0, sem0),
                (src_v1, dst_v1, rows_v1, sem1))

        def prefetch(j, b):
            src_v, dst_v, rows_v, sem = bufs[b]
            base = wid * epw + j * ch
            pltpu.sync_copy(src.at[pl.ds(base, ch)], src_v)
            pltpu.sync_copy(dst.at[pl.ds(base, ch)], dst_v)
            pltpu.async_copy(table.at[src_v], rows_v, sem)

        def finish(b):
            src_v, dst_v, rows_v, sem = bufs[b]
            pltpu.make_async_copy(table.at[src_v], rows_v, sem).wait()
            pltpu.sync_copy(rows_v, agg_sh.at[dst_v], add=True)

        _fill_f32(zrow_v, ZR, d, 0.0)

        for z in range(nz):
            r0 = (s * nz + z) * ZR
            pltpu.sync_copy(zrow_v, agg_sh.at[pl.ds(r0, ZR)])
        plsc.subcore_barrier()

        prefetch(0, 0)
        prefetch(1, 1)

        def pair(p, _):
            j = 2 * p
            finish(0)

            @pl.when(j + 2 < n_chunks)
            def _():
                prefetch(j + 2, 0)

            finish(1)

            @pl.when(j + 3 < n_chunks)
            def _():
                prefetch(j + 3, 1)

            return 0

        lax.fori_loop(0, n_chunks // 2, pair, 0, unroll=False)
        plsc.subcore_barrier()

        r0 = s * rpt
        pltpu.sync_copy(agg_sh.at[pl.ds(r0, rpt)],
                        out_agg.at[pl.ds(c * n_acc + r0, rpt)])

    return pl.kernel(body, out_type=out_type, mesh=mesh,
                     scratch_types=scratch)


def _make_cnt(n_acc, e_pad, cw=128, ch=CH):
    epw = e_pad // NW
    n_chunks = epw // ch
    rpt = n_acc // NS
    nz = rpt // ZR
    mesh = plsc.VectorSubcoreMesh(core_axis_name="c", subcore_axis_name="s")

    def body(dst, out_cnt, dst_v, ones_v, zcnt_v, cnt_sh):
        c = lax.axis_index("c")
        s = lax.axis_index("s")
        wid = s * NC + c

        _fill_f32(ones_v, ch, cw, 1.0)
        _fill_f32(zcnt_v, ZR, cw, 0.0)
        for z in range(nz):
            pltpu.sync_copy(zcnt_v, cnt_sh.at[pl.ds((s * nz + z) * ZR, ZR)])
        plsc.subcore_barrier()

        def chunk(j, _):
            base = wid * epw + j * ch
            pltpu.sync_copy(dst.at[pl.ds(base, ch)], dst_v)
            pltpu.sync_copy(ones_v, cnt_sh.at[dst_v], add=True)
            return 0

        lax.fori_loop(0, n_chunks, chunk, 0, unroll=False)
        plsc.subcore_barrier()

        r0 = s * rpt
        pltpu.sync_copy(cnt_sh.at[pl.ds(r0, rpt)],
                        out_cnt.at[pl.ds(c * n_acc + r0, rpt)])

    return pl.kernel(
        body,
        out_type=jax.ShapeDtypeStruct((NC * n_acc, cw), jnp.float32),
        mesh=mesh,
        scratch_types=[
            pltpu.VMEM((ch,), jnp.int32),
            pltpu.VMEM((ch, cw), jnp.float32),
            pltpu.VMEM((ZR, cw), jnp.float32),
            pltpu.VMEM_SHARED((n_acc, cw), jnp.float32),
        ],
    )


def _make_pair_gather(d, b_pad):
    bpw = b_pad // NW
    n_chunks = bpw // CH
    mesh = plsc.VectorSubcoreMesh(core_axis_name="c", subcore_axis_name="s")

    assert n_chunks % 2 == 0

    def body(table, idx, out, idx_v0, rows_v0, idx_v1, rows_v1, sem0, sem1):
        c = lax.axis_index("c")
        s = lax.axis_index("s")
        wid = s * NC + c
        bufs = ((idx_v0, rows_v0, sem0), (idx_v1, rows_v1, sem1))

        def prefetch(j, b):
            idx_v, rows_v, sem = bufs[b]
            pltpu.sync_copy(idx.at[pl.ds(wid * bpw + j * CH, CH)], idx_v)
            pltpu.async_copy(table.at[idx_v], rows_v, sem)

        def finish(j, b):
            idx_v, rows_v, sem = bufs[b]
            pltpu.make_async_copy(table.at[idx_v], rows_v, sem).wait()
            pltpu.sync_copy(rows_v, out.at[pl.ds(wid * bpw + j * CH, CH)])

        prefetch(0, 0)
        prefetch(1, 1)

        def pair(p, _):
            j = 2 * p
            finish(j, 0)

            @pl.when(j + 2 < n_chunks)
            def _():
                prefetch(j + 2, 0)

            finish(j + 1, 1)

            @pl.when(j + 3 < n_chunks)
            def _():
                prefetch(j + 3, 1)

            return 0

        lax.fori_loop(0, n_chunks // 2, pair, 0, unroll=False)

    return pl.kernel(
        body,
        out_type=jax.ShapeDtypeStruct((b_pad, d), jnp.float32),
        mesh=mesh,
        scratch_types=[
            pltpu.VMEM((CH,), jnp.int32),
            pltpu.VMEM((CH, d), jnp.float32),
            pltpu.VMEM((CH,), jnp.int32),
            pltpu.VMEM((CH, d), jnp.float32),
            pltpu.SemaphoreType.DMA,
            pltpu.SemaphoreType.DMA,
        ],
    )


def _node1_body(aggp, cntp, x, wlt, wrt, bl, out):
    cnt = cntp[0, :, 0:1] + cntp[1, :, 0:1]
    mean = (aggp[0] + aggp[1]) / jnp.maximum(cnt, 1.0)
    h = (jnp.dot(mean, wlt[...], preferred_element_type=jnp.float32)
         + jnp.dot(x[...], wrt[...], preferred_element_type=jnp.float32)
         + bl[...])
    out[...] = jnp.maximum(h, 0.0)


def _node2_body(aggp, cntp, h1, wlt, wrt, bl, p1t, p1b, p2t, p2b,
                m1at, m1bt, m1b, out):
    cnt = cntp[0, :, 0:1] + cntp[1, :, 0:1]
    mean = (aggp[0] + aggp[1]) / jnp.maximum(cnt, 1.0)
    h2 = (jnp.dot(mean, wlt[...], preferred_element_type=jnp.float32)
          + jnp.dot(h1[...], wrt[...], preferred_element_type=jnp.float32)
          + bl[...])
    h2 = jnp.maximum(h2, 0.0)
    hp = jnp.dot(h2, p1t[...], preferred_element_type=jnp.float32) + p1b[...]
    hp = jnp.dot(hp, p2t[...], preferred_element_type=jnp.float32) + p2b[...]
    out[0] = jnp.dot(hp, m1at[...], preferred_element_type=jnp.float32) + m1b[...]
    out[1] = jnp.dot(hp, m1bt[...], preferred_element_type=jnp.float32)


def _edge_body(g, m2t, m2b, m3t, m3b, m4r, m4b, out):
    e = jnp.maximum(g[0] + g[1], 0.0)
    e = jnp.maximum(
        jnp.dot(e, m2t[...], preferred_element_type=jnp.float32) + m2b[...], 0.0)
    e = jnp.maximum(
        jnp.dot(e, m3t[...], preferred_element_type=jnp.float32) + m3b[...], 0.0)
    o = jnp.sum(e * m4r[...], axis=1, keepdims=True) + m4b[...]
    out[...] = jnp.maximum(o, 0.0)


def _round_up(a, b):
    return (a + b - 1) // b * b


def kernel(x, edge_attr, edge_index, predict_edge_index, Wl1, bl1, Wr1,
           Wl2, bl2, Wr2, P1, p1b, P2, p2b, M1, m1b, M2, m2b, M3, m3b,
           M4, m4b):
    del edge_attr
    n, d = x.shape
    h = Wl1.shape[0]
    o = P2.shape[0]
    e = edge_index.shape[1]
    ep = predict_edge_index.shape[1]
    f32 = jnp.float32

    n_acc = _round_up(n + 1, NS * ZR)
    e_pad = _round_up(e, NW * 96 * 2)
    ep_pad = _round_up(ep, NW * CH)
    b_pad = 2 * ep_pad

    x_pad = jnp.pad(x, ((0, n_acc - n), (0, 0)))
    efill = jnp.arange(e_pad - e, dtype=jnp.int32) % n
    src = jnp.concatenate([edge_index[0], efill])
    dst = jnp.pad(edge_index[1], (0, e_pad - e), constant_values=n)
    pfill = jnp.arange(ep_pad - ep, dtype=jnp.int32) % n
    pe0 = jnp.concatenate([predict_edge_index[0], pfill])
    pe1 = jnp.concatenate([predict_edge_index[1], pfill])
    pair_idx = jnp.concatenate([pe0, pe1 + n_acc])

    wl1t = Wl1.T
    wr1t = Wr1.T
    wl2t = Wl2.T
    wr2t = Wr2.T
    p1t = P1.T
    p2t = P2.T
    m1at = M1[:, :o].T
    m1bt = M1[:, o:].T
    m2t = M2.T
    m3t = M3.T
    m4r = M4
    bl1r = bl1.reshape(1, h)
    bl2r = bl2.reshape(1, h)
    p1br = p1b.reshape(1, h)
    p2br = p2b.reshape(1, o)
    m1br = m1b.reshape(1, o)
    m2br = m2b.reshape(1, o)
    m3br = m3b.reshape(1, o)
    m4br = m4b.reshape(1, 1)

    aggp1 = _make_agg(n_acc, d, e_pad, ch=96)(x, src, dst)
    aggp1 = aggp1.reshape(NC, n_acc, d)
    cntp = _make_cnt(n_acc, e_pad, ch=96)(dst)
    cntp = cntp.reshape(NC, n_acc, 128)

    br = 2048
    grid_n = n_acc // br
    wspec = pl.BlockSpec((d, h), lambda i: (0, 0))
    bspec = lambda cols: pl.BlockSpec((1, cols), lambda i: (0, 0))
    h1 = pl.pallas_call(
        _node1_body,
        grid=(grid_n,),
        in_specs=[
            pl.BlockSpec((2, br, d), lambda i: (0, i, 0)),
            pl.BlockSpec((2, br, 128), lambda i: (0, i, 0)),
            pl.BlockSpec((br, d), lambda i: (i, 0)),
            wspec, wspec, bspec(h),
        ],
        out_specs=pl.BlockSpec((br, h), lambda i: (i, 0)),
        out_shape=jax.ShapeDtypeStruct((n_acc, h), f32),
    )(aggp1, cntp, x_pad, wl1t, wr1t, bl1r)

    aggp2 = _make_agg(n_acc, h, e_pad, ch=96)(h1, src, dst)
    aggp2 = aggp2.reshape(NC, n_acc, h)

    ab = pl.pallas_call(
        _node2_body,
        grid=(grid_n,),
        in_specs=[
            pl.BlockSpec((2, br, h), lambda i: (0, i, 0)),
            pl.BlockSpec((2, br, 128), lambda i: (0, i, 0)),
            pl.BlockSpec((br, h), lambda i: (i, 0)),
            wspec, wspec, bspec(h),
            wspec, bspec(h), wspec, bspec(o),
            wspec, wspec, bspec(o),
        ],
        out_specs=pl.BlockSpec((2, br, o), lambda i: (0, i, 0)),
        out_shape=jax.ShapeDtypeStruct((2, n_acc, o), f32),
    )(aggp2, cntp, h1, wl2t, wr2t, bl2r, p1t, p1br, p2t, p2br,
      m1at, m1bt, m1br)
    tab = ab.reshape(2 * n_acc, o)

    g = _make_pair_gather(o, b_pad)(tab, pair_idx)
    g2 = g.reshape(2, ep_pad, o)

    bre = 2048
    grid_e = ep_pad // bre
    espec = pl.BlockSpec((o, o), lambda i: (0, 0))
    ebspec = lambda cols: pl.BlockSpec((1, cols), lambda i: (0, 0))
    out = pl.pallas_call(
        _edge_body,
        grid=(grid_e,),
        in_specs=[
            pl.BlockSpec((2, bre, o), lambda i: (0, i, 0)),
            espec, ebspec(o), espec, ebspec(o), ebspec(o), ebspec(1),
        ],
        out_specs=pl.BlockSpec((bre, 1), lambda i: (i, 0)),
        out_shape=jax.ShapeDtypeStruct((ep_pad, 1), f32),
    )(g2, m2t, m2br, m3t, m3br, m4r, m4br)

    return out[:ep]
